# baseline (speedup 1.0000x reference)
import functools

import jax
import jax.numpy as jnp
from jax import lax
from jax.experimental import pallas as pl
from jax.experimental.pallas import tpu as pltpu
from jax.experimental.pallas import tpu_sc as plsc

N_NODES = 10000
N_EDGES = 320000
D_FEAT = 128

NC = 2
NS = 16
L = 16
NW = NC * NS
EPW = N_EDGES // NW
C = 80
NCHUNK = EPW // C
G = C // L
NB = 4
DU = 8
W = D_FEAT // 2


def _body(z_hbm, src_hbm, dst_hbm, out_hbm,
          z_sh, idx_s_v, idx_d_v, rows_s, rows_d, outb,
          sems_s, sems_d, sems_o):
    cid = lax.axis_index("c")
    sid = lax.axis_index("s")
    wid = sid * NC + cid
    ebase = wid * EPW

    zrows = 1000

    @pl.when(sid < N_NODES // zrows)
    def _stage():
        pltpu.sync_copy(z_hbm.at[pl.ds(sid * zrows, zrows)],
                        z_sh.at[pl.ds(sid * zrows, zrows)])

    pltpu.sync_copy(src_hbm.at[pl.ds(ebase, EPW)], idx_s_v)
    pltpu.sync_copy(dst_hbm.at[pl.ds(ebase, EPW)], idx_d_v)
    plsc.subcore_barrier()

    lanes = lax.iota(jnp.int32, L)

    def start(g, b):
        off = g * C
        pltpu.async_copy(z_sh.at[idx_s_v.at[pl.ds(off, C)]],
                         rows_s.at[b], sems_s.at[b])
        pltpu.async_copy(z_sh.at[idx_d_v.at[pl.ds(off, C)]],
                         rows_d.at[b], sems_d.at[b])

    def wait_rows(b):
        pltpu.make_async_copy(z_hbm.at[pl.ds(0, C)],
                              rows_s.at[b], sems_s.at[b]).wait()
        pltpu.make_async_copy(z_hbm.at[pl.ds(0, C)],
                              rows_d.at[b], sems_d.at[b]).wait()

    def compute(g, b):
        rs = rows_s.at[b]
        rd = rows_d.at[b]
        ob = outb.at[b]

        @pl.when(g >= NB)
        def _drain():
            pltpu.make_async_copy(ob, out_hbm.at[pl.ds(0, C)],
                                  sems_o.at[b]).wait()


        def estep(blk, _):
            row_ids = blk * L + lanes

            def dstep(d0, carry):
                acc, dl = carry
                for _ in range(DU):
                    a = plsc.load_gather(rs, [row_ids, dl])
                    b2 = plsc.load_gather(rd, [row_ids, dl])
                    prod = (plsc.bitcast(a, jnp.bfloat16)
                            * plsc.bitcast(b2, jnp.bfloat16))
                    pa, pb = plsc.unpack(
                        prod, format=plsc.PackFormat.INTERLEAVED,
                        preferred_element_type=jnp.float32)
                    acc = acc + pa + pb
                    dl = (dl + 1) & (W - 1)
                return acc, dl

            acc, _dl = lax.fori_loop(0, W // DU, dstep,
                                     (jnp.zeros((L,), jnp.float32), lanes))
            ob[pl.ds(blk * L, L)] = 1.0 / (1.0 + jnp.exp(-acc))
            return _

        lax.fori_loop(0, G, estep, 0)
        pltpu.async_copy(ob, out_hbm.at[pl.ds(ebase + g * C, C)],
                         sems_o.at[b])

    for b in range(NB - 1):
        start(b, b)

    def quad(i, carry):
        g0 = i * NB
        for u in range(NB):
            g = g0 + u

            @pl.when(g + NB - 1 < NCHUNK)
            def _ahead():
                start(g + NB - 1, (g + NB - 1) % NB)

            wait_rows(u)
            compute(g, u)
        return carry

    lax.fori_loop(0, NCHUNK // NB, quad, 0)
    g_last = NCHUNK - 1
    wait_rows(g_last % NB)
    compute(g_last, g_last % NB)

    for b in range(NB):
        pltpu.make_async_copy(outb.at[b], out_hbm.at[pl.ds(0, C)],
                              sems_o.at[b]).wait()


_mesh = plsc.VectorSubcoreMesh(
    core_axis_name="c", subcore_axis_name="s", num_cores=NC, num_subcores=NS)

_call = functools.partial(
    pl.kernel,
    out_type=jax.ShapeDtypeStruct((N_EDGES,), jnp.float32),
    mesh=_mesh,
    scratch_types=[
        pltpu.VMEM_SHARED((N_NODES, W), jnp.int32),
        pltpu.VMEM((EPW,), jnp.int32),
        pltpu.VMEM((EPW,), jnp.int32),
        pltpu.VMEM((NB, C, W), jnp.int32),
        pltpu.VMEM((NB, C, W), jnp.int32),
        pltpu.VMEM((NB, C), jnp.float32),
        pltpu.SemaphoreType.DMA((NB,)),
        pltpu.SemaphoreType.DMA((NB,)),
        pltpu.SemaphoreType.DMA((NB,)),
    ],
    compiler_params=pltpu.CompilerParams(needs_layout_passes=False,
                                         use_tc_tiling_on_sc=False),
)(_body)


def kernel(z, edge_index):
    src = edge_index[0]
    dst = edge_index[1]
    z_packed = lax.bitcast_convert_type(
        z.astype(jnp.bfloat16).reshape(N_NODES, W, 2), jnp.int32)
    return _call(z_packed, src, dst)


# speedup vs baseline: 1.4264x; 1.4264x over previous
import functools

import jax
import jax.numpy as jnp
from jax import lax
from jax.experimental import pallas as pl
from jax.experimental.pallas import tpu as pltpu
from jax.experimental.pallas import tpu_sc as plsc

N_NODES = 10000
N_EDGES = 320000
D_FEAT = 128

NC = 2
NS = 16
L = 16
NW = NC * NS
EPW = N_EDGES // NW
C = 80
NCHUNK = EPW // C
G = C // L
NB = 4
DU = 8
W = D_FEAT // 2


def _body(z_hbm, src_hbm, dst_hbm, out_hbm,
          z_sh, idx_s_v, idx_d_v, rows_s, rows_d, outb,
          sems_s, sems_d, sems_o):
    cid = lax.axis_index("c")
    sid = lax.axis_index("s")
    wid = sid * NC + cid
    ebase = wid * EPW

    zrows = 2000

    @pl.when(sid < N_NODES // zrows)
    def _stage():
        pltpu.sync_copy(z_hbm.at[pl.ds(sid * zrows, zrows)],
                        z_sh.at[pl.ds(sid * zrows, zrows)])

    pltpu.sync_copy(src_hbm.at[pl.ds(ebase, EPW)], idx_s_v)
    pltpu.sync_copy(dst_hbm.at[pl.ds(ebase, EPW)], idx_d_v)
    plsc.subcore_barrier()

    lanes = lax.iota(jnp.int32, L)

    def start(g, b):
        off = g * C
        pltpu.async_copy(z_sh.at[idx_s_v.at[pl.ds(off, C)]],
                         rows_s.at[b], sems_s.at[b])
        pltpu.async_copy(z_sh.at[idx_d_v.at[pl.ds(off, C)]],
                         rows_d.at[b], sems_d.at[b])

    def wait_rows(b):
        pltpu.make_async_copy(z_hbm.at[pl.ds(0, C)],
                              rows_s.at[b], sems_s.at[b]).wait()
        pltpu.make_async_copy(z_hbm.at[pl.ds(0, C)],
                              rows_d.at[b], sems_d.at[b]).wait()

    def compute(g, b):
        rs = rows_s.at[b]
        rd = rows_d.at[b]
        ob = outb.at[b]

        @pl.when(g >= NB)
        def _drain():
            pltpu.make_async_copy(ob, out_hbm.at[pl.ds(0, C)],
                                  sems_o.at[b]).wait()


        def estep(blk, _):
            e_base = blk * L

            def dot16(e):
                p = []
                for j in range(D_FEAT // (2 * L)):
                    s2 = rs[e, pl.ds(j * 2 * L, 2 * L)]
                    d2 = rd[e, pl.ds(j * 2 * L, 2 * L)]
                    pa, pb = plsc.unpack(
                        s2 * d2, format=plsc.PackFormat.INTERLEAVED,
                        preferred_element_type=jnp.float32)
                    p.append(pa + pb)
                while len(p) > 1:
                    p = [a + b for a, b in zip(p[::2], p[1::2])]
                return jnp.sum(p[0])

            def eight_edges(u, res):
                e = e_base + u * 8
                for q in range(8):
                    res = jnp.where(lanes == u * 8 + q, dot16(e + q), res)
                return res

            res = lax.fori_loop(0, L // 8, eight_edges,
                                jnp.zeros((L,), jnp.float32))
            ob[pl.ds(e_base, L)] = 1.0 / (1.0 + jnp.exp(-res))
            return _

        lax.fori_loop(0, G, estep, 0)
        pltpu.async_copy(ob, out_hbm.at[pl.ds(ebase + g * C, C)],
                         sems_o.at[b])

    for b in range(NB - 1):
        start(b, b)

    def quad(i, carry):
        g0 = i * NB
        for u in range(NB):
            g = g0 + u

            @pl.when(g + NB - 1 < NCHUNK)
            def _ahead():
                start(g + NB - 1, (g + NB - 1) % NB)

            wait_rows(u)
            compute(g, u)
        return carry

    lax.fori_loop(0, NCHUNK // NB, quad, 0)
    g_last = NCHUNK - 1
    wait_rows(g_last % NB)
    compute(g_last, g_last % NB)

    for b in range(NB):
        pltpu.make_async_copy(outb.at[b], out_hbm.at[pl.ds(0, C)],
                              sems_o.at[b]).wait()


_mesh = plsc.VectorSubcoreMesh(
    core_axis_name="c", subcore_axis_name="s", num_cores=NC, num_subcores=NS)

_call = functools.partial(
    pl.kernel,
    out_type=jax.ShapeDtypeStruct((N_EDGES,), jnp.float32),
    mesh=_mesh,
    scratch_types=[
        pltpu.VMEM_SHARED((N_NODES, D_FEAT), jnp.bfloat16),
        pltpu.VMEM((EPW,), jnp.int32),
        pltpu.VMEM((EPW,), jnp.int32),
        pltpu.VMEM((NB, C, D_FEAT), jnp.bfloat16),
        pltpu.VMEM((NB, C, D_FEAT), jnp.bfloat16),
        pltpu.VMEM((NB, C), jnp.float32),
        pltpu.SemaphoreType.DMA((NB,)),
        pltpu.SemaphoreType.DMA((NB,)),
        pltpu.SemaphoreType.DMA((NB,)),
    ],
    compiler_params=pltpu.CompilerParams(needs_layout_passes=False,
                                         use_tc_tiling_on_sc=False),
)(_body)


def kernel(z, edge_index):
    src = edge_index[0]
    dst = edge_index[1]
    return _call(z.astype(jnp.bfloat16), src, dst)


# speedup vs baseline: 1.5692x; 1.1001x over previous
import functools

import jax
import jax.numpy as jnp
from jax import lax
from jax.experimental import pallas as pl
from jax.experimental.pallas import tpu as pltpu
from jax.experimental.pallas import tpu_sc as plsc

N_NODES = 10000
N_EDGES = 320000
D_FEAT = 128

NC = 2
NS = 16
L = 16
NW = NC * NS
EPW = N_EDGES // NW
C = 80
NCHUNK = EPW // C
G = C // L
NB = 4
DU = 8
W = D_FEAT // 2


def _body(z_hbm, src_hbm, dst_hbm, out_hbm,
          z_sh, idx_s_v, idx_d_v, rows_s, rows_d, outb,
          sems_s, sems_d, sems_o):
    cid = lax.axis_index("c")
    sid = lax.axis_index("s")
    wid = sid * NC + cid
    ebase = wid * EPW

    zrows = 2000

    @pl.when(sid < N_NODES // zrows)
    def _stage():
        pltpu.sync_copy(z_hbm.at[pl.ds(sid * zrows, zrows)],
                        z_sh.at[pl.ds(sid * zrows, zrows)])

    pltpu.sync_copy(src_hbm.at[pl.ds(ebase, EPW)], idx_s_v)
    pltpu.sync_copy(dst_hbm.at[pl.ds(ebase, EPW)], idx_d_v)
    plsc.subcore_barrier()

    lanes = lax.iota(jnp.int32, L)

    def start(g, b):
        off = g * C
        pltpu.async_copy(z_sh.at[idx_s_v.at[pl.ds(off, C)]],
                         rows_s.at[b], sems_s.at[b])
        pltpu.async_copy(z_sh.at[idx_d_v.at[pl.ds(off, C)]],
                         rows_d.at[b], sems_d.at[b])

    def wait_rows(b):
        pltpu.make_async_copy(z_hbm.at[pl.ds(0, C)],
                              rows_s.at[b], sems_s.at[b]).wait()
        pltpu.make_async_copy(z_hbm.at[pl.ds(0, C)],
                              rows_d.at[b], sems_d.at[b]).wait()

    def compute(g, b):
        rs = rows_s.at[b]
        rd = rows_d.at[b]
        ob = outb.at[b]

        @pl.when(g >= NB)
        def _drain():
            pltpu.make_async_copy(ob, out_hbm.at[pl.ds(0, C)],
                                  sems_o.at[b]).wait()


        def estep(blk, _):
            e_base = blk * L

            def dot16(e):
                p = []
                for j in range(D_FEAT // (2 * L)):
                    s2 = rs[e, pl.ds(j * 2 * L, 2 * L)]
                    d2 = rd[e, pl.ds(j * 2 * L, 2 * L)]
                    pa, pb = plsc.unpack(
                        s2 * d2, format=plsc.PackFormat.INTERLEAVED,
                        preferred_element_type=jnp.float32)
                    p.append(pa + pb)
                while len(p) > 1:
                    p = [a + b for a, b in zip(p[::2], p[1::2])]
                return jnp.sum(p[0])

            def four_edges(u, res):
                e = e_base + u * 4
                for q in range(4):
                    res = jnp.where(lanes == u * 4 + q, dot16(e + q), res)
                return res

            res = lax.fori_loop(0, L // 4, four_edges,
                                jnp.zeros((L,), jnp.float32))
            ob[pl.ds(e_base, L)] = 1.0 / (1.0 + jnp.exp(-res))
            return _

        lax.fori_loop(0, G, estep, 0)
        pltpu.async_copy(ob, out_hbm.at[pl.ds(ebase + g * C, C)],
                         sems_o.at[b])

    for b in range(NB - 1):
        start(b, b)

    def quad(i, carry):
        g0 = i * NB
        for u in range(NB):
            g = g0 + u

            @pl.when(g + NB - 1 < NCHUNK)
            def _ahead():
                start(g + NB - 1, (g + NB - 1) % NB)

            wait_rows(u)
            compute(g, u)
        return carry

    lax.fori_loop(0, NCHUNK // NB, quad, 0)
    g_last = NCHUNK - 1
    wait_rows(g_last % NB)
    compute(g_last, g_last % NB)

    for b in range(NB):
        pltpu.make_async_copy(outb.at[b], out_hbm.at[pl.ds(0, C)],
                              sems_o.at[b]).wait()


_mesh = plsc.VectorSubcoreMesh(
    core_axis_name="c", subcore_axis_name="s", num_cores=NC, num_subcores=NS)

_call = functools.partial(
    pl.kernel,
    out_type=jax.ShapeDtypeStruct((N_EDGES,), jnp.float32),
    mesh=_mesh,
    scratch_types=[
        pltpu.VMEM_SHARED((N_NODES, D_FEAT), jnp.bfloat16),
        pltpu.VMEM((EPW,), jnp.int32),
        pltpu.VMEM((EPW,), jnp.int32),
        pltpu.VMEM((NB, C, D_FEAT), jnp.bfloat16),
        pltpu.VMEM((NB, C, D_FEAT), jnp.bfloat16),
        pltpu.VMEM((NB, C), jnp.float32),
        pltpu.SemaphoreType.DMA((NB,)),
        pltpu.SemaphoreType.DMA((NB,)),
        pltpu.SemaphoreType.DMA((NB,)),
    ],
    compiler_params=pltpu.CompilerParams(needs_layout_passes=False,
                                         use_tc_tiling_on_sc=False),
)(_body)


def kernel(z, edge_index):
    src = edge_index[0]
    dst = edge_index[1]
    return _call(z.astype(jnp.bfloat16), src, dst)


# speedup vs baseline: 1.5698x; 1.0004x over previous
import functools

import jax
import jax.numpy as jnp
from jax import lax
from jax.experimental import pallas as pl
from jax.experimental.pallas import tpu as pltpu
from jax.experimental.pallas import tpu_sc as plsc

N_NODES = 10000
N_EDGES = 320000
D_FEAT = 128

NC = 2
NS = 16
L = 16
NW = NC * NS
EPW = N_EDGES // NW
C = 80
NCHUNK = EPW // C
G = C // L
NB = 4
DU = 8
W = D_FEAT // 2


def _body(z_hbm, src_hbm, dst_hbm, out_hbm,
          z_sh, idx_s_v, idx_d_v, rows_s, rows_d, outb,
          sems_s, sems_d, sems_o):
    cid = lax.axis_index("c")
    sid = lax.axis_index("s")
    wid = sid * NC + cid
    ebase = wid * EPW

    zrows = 2000

    @pl.when(sid < N_NODES // zrows)
    def _stage():
        pltpu.sync_copy(z_hbm.at[pl.ds(sid * zrows, zrows)],
                        z_sh.at[pl.ds(sid * zrows, zrows)])

    pltpu.sync_copy(src_hbm.at[pl.ds(ebase, EPW)], idx_s_v)
    pltpu.sync_copy(dst_hbm.at[pl.ds(ebase, EPW)], idx_d_v)
    plsc.subcore_barrier()

    lanes = lax.iota(jnp.int32, L)

    def start(g, b):
        off = g * C
        pltpu.async_copy(z_sh.at[idx_s_v.at[pl.ds(off, C)]],
                         rows_s.at[b], sems_s.at[b])
        pltpu.async_copy(z_sh.at[idx_d_v.at[pl.ds(off, C)]],
                         rows_d.at[b], sems_d.at[b])

    def wait_rows(b):
        pltpu.make_async_copy(z_hbm.at[pl.ds(0, C)],
                              rows_s.at[b], sems_s.at[b]).wait()
        pltpu.make_async_copy(z_hbm.at[pl.ds(0, C)],
                              rows_d.at[b], sems_d.at[b]).wait()

    def compute(g, b):
        rs = rows_s.at[b]
        rd = rows_d.at[b]
        ob = outb.at[b]

        @pl.when(g >= NB)
        def _drain():
            pltpu.make_async_copy(ob, out_hbm.at[pl.ds(0, C)],
                                  sems_o.at[b]).wait()


        @plsc.parallel_loop(0, G)
        def estep(blk):
            e_base = blk * L

            def dot16(e):
                p = []
                for j in range(D_FEAT // (2 * L)):
                    s2 = rs[e, pl.ds(j * 2 * L, 2 * L)]
                    d2 = rd[e, pl.ds(j * 2 * L, 2 * L)]
                    pa, pb = plsc.unpack(
                        s2 * d2, format=plsc.PackFormat.INTERLEAVED,
                        preferred_element_type=jnp.float32)
                    p.append(pa + pb)
                while len(p) > 1:
                    p = [a + b for a, b in zip(p[::2], p[1::2])]
                return jnp.sum(p[0])

            def four_edges(u, res):
                e = e_base + u * 4
                for q in range(4):
                    res = jnp.where(lanes == u * 4 + q, dot16(e + q), res)
                return res

            res = lax.fori_loop(0, L // 4, four_edges,
                                jnp.zeros((L,), jnp.float32))
            ob[pl.ds(e_base, L)] = 1.0 / (1.0 + jnp.exp(-res))
        pltpu.async_copy(ob, out_hbm.at[pl.ds(ebase + g * C, C)],
                         sems_o.at[b])

    for b in range(NB - 1):
        start(b, b)

    def quad(i, carry):
        g0 = i * NB
        for u in range(NB):
            g = g0 + u

            @pl.when(g + NB - 1 < NCHUNK)
            def _ahead():
                start(g + NB - 1, (g + NB - 1) % NB)

            wait_rows(u)
            compute(g, u)
        return carry

    lax.fori_loop(0, NCHUNK // NB, quad, 0)
    g_last = NCHUNK - 1
    wait_rows(g_last % NB)
    compute(g_last, g_last % NB)

    for b in range(NB):
        pltpu.make_async_copy(outb.at[b], out_hbm.at[pl.ds(0, C)],
                              sems_o.at[b]).wait()


_mesh = plsc.VectorSubcoreMesh(
    core_axis_name="c", subcore_axis_name="s", num_cores=NC, num_subcores=NS)

_call = functools.partial(
    pl.kernel,
    out_type=jax.ShapeDtypeStruct((N_EDGES,), jnp.float32),
    mesh=_mesh,
    scratch_types=[
        pltpu.VMEM_SHARED((N_NODES, D_FEAT), jnp.bfloat16),
        pltpu.VMEM((EPW,), jnp.int32),
        pltpu.VMEM((EPW,), jnp.int32),
        pltpu.VMEM((NB, C, D_FEAT), jnp.bfloat16),
        pltpu.VMEM((NB, C, D_FEAT), jnp.bfloat16),
        pltpu.VMEM((NB, C), jnp.float32),
        pltpu.SemaphoreType.DMA((NB,)),
        pltpu.SemaphoreType.DMA((NB,)),
        pltpu.SemaphoreType.DMA((NB,)),
    ],
    compiler_params=pltpu.CompilerParams(needs_layout_passes=False,
                                         use_tc_tiling_on_sc=False),
)(_body)


def kernel(z, edge_index):
    src = edge_index[0]
    dst = edge_index[1]
    return _call(z.astype(jnp.bfloat16), src, dst)
